# baseline (speedup 1.0000x reference)
import jax
import jax.numpy as jnp
from jax.experimental import pallas as pl
from jax.experimental.pallas import tpu as pltpu

_STRIDES = [8, 16, 32, 64, 128]
_THRESHOLD = 0.99
_HWS = [64, 32, 16, 8, 4]
_NLOC = [hw * hw for hw in _HWS]
_OFFS = [0, 4096, 5120, 5376, 5440]
_TOT = 5456
_C = 80
_OUTC = 87


def _cls_placement():
    r = jax.lax.broadcasted_iota(jnp.int32, (_OUTC, _C), 0)
    c = jax.lax.broadcasted_iota(jnp.int32, (_OUTC, _C), 1)
    return (r == c + 2).astype(jnp.float32)


def _small_placement_t():
    r = jax.lax.broadcasted_iota(jnp.int32, (_OUTC, 8), 0)
    c = jax.lax.broadcasted_iota(jnp.int32, (_OUTC, 8), 1)
    e = (r == c + 82) & (c < 5)
    e = e | ((c == 5) & (r == 0)) | ((c == 6) & (r == 1))
    return e.astype(jnp.float32)


def _body(c0, c1, c2, c3, c4, b0, b1, b2, b3, b4, t0, t1, t2, t3, t4,
          out_ref, small_ref):
    cls_refs = [c0, c1, c2, c3, c4]
    bbox_refs = [b0, b1, b2, b3, b4]
    ctr_refs = [t0, t1, t2, t3, t4]
    dt = (((1,), (1,)), ((), ()))
    ds = (((1,), (0,)), ((), ()))
    small_ref[7:8, :] = jnp.zeros((1, _NLOC[0]), jnp.float32)
    for l in range(5):
        m = _NLOC[l]
        hw = _HWS[l]
        v = cls_refs[l][0]
        ind = (v > _THRESHOLD).astype(jnp.float32)
        srow = jax.lax.dot_general(jnp.ones((1, _C), jnp.float32), ind, dt,
                                   preferred_element_type=jnp.float32)
        mask = (srow > 0.0).astype(jnp.float32)
        big = jax.lax.dot_general(_cls_placement(), v, dt,
                                  preferred_element_type=jnp.float32)
        im = jax.lax.broadcasted_iota(jnp.int32, (1, m), 1)
        small_ref[0:4, 0:m] = bbox_refs[l][0]
        small_ref[4:5, 0:m] = ctr_refs[l][0]
        small_ref[5:6, 0:m] = ((im % hw) * _STRIDES[l]).astype(jnp.float32)
        small_ref[6:7, 0:m] = ((im // hw) * _STRIDES[l]).astype(jnp.float32)
        t = jax.lax.dot_general(_small_placement_t(), small_ref[:, 0:m], ds,
                                preferred_element_type=jnp.float32)
        out_ref[0, :, pl.ds(_OFFS[l], m)] = (big + t) * mask


def kernel(cls_scores_0, cls_scores_1, cls_scores_2, cls_scores_3, cls_scores_4,
           bbox_preds_0, bbox_preds_1, bbox_preds_2, bbox_preds_3, bbox_preds_4,
           centernesses_0, centernesses_1, centernesses_2, centernesses_3,
           centernesses_4):
    n = cls_scores_0.shape[0]
    cls_l = [cls_scores_0, cls_scores_1, cls_scores_2, cls_scores_3, cls_scores_4]
    bbox_l = [bbox_preds_0, bbox_preds_1, bbox_preds_2, bbox_preds_3, bbox_preds_4]
    ctr_l = [centernesses_0, centernesses_1, centernesses_2, centernesses_3,
             centernesses_4]
    args = []
    specs = []
    for l in range(5):
        m = _NLOC[l]
        args.append(jnp.transpose(cls_l[l], (0, 2, 3, 1)).reshape(n, m, _C))
        specs.append(pl.BlockSpec((1, m, _C), lambda i: (i, 0, 0)))
    for lst, ch in ((bbox_l, 4), (ctr_l, 1)):
        for l in range(5):
            args.append(lst[l].reshape(n, ch, _NLOC[l]))
            specs.append(pl.BlockSpec((1, ch, _NLOC[l]), lambda i: (i, 0, 0)))
    out = pl.pallas_call(
        _body,
        grid=(n,),
        in_specs=specs,
        out_specs=pl.BlockSpec((1, _OUTC, _TOT), lambda i: (i, 0, 0)),
        out_shape=jax.ShapeDtypeStruct((n, _OUTC, _TOT), jnp.float32),
        scratch_shapes=[pltpu.VMEM((8, _NLOC[0]), jnp.float32)],
        compiler_params=pltpu.CompilerParams(
            dimension_semantics=("parallel",)),
    )(*args)
    return jnp.transpose(out, (0, 2, 1))


# speedup vs baseline: 1.0451x; 1.0451x over previous
import jax
import jax.numpy as jnp
from jax.experimental import pallas as pl
from jax.experimental.pallas import tpu as pltpu

_STRIDES = [8, 16, 32, 64, 128]
_THRESHOLD = 0.99
_HWS = [64, 32, 16, 8, 4]
_NLOC = [hw * hw for hw in _HWS]
_OFFS = [0, 4096, 5120, 5376, 5440]
_TOT = 5456
_C = 80
_OUTC = 87
_IPB = 2


def _cls_placement():
    r = jax.lax.broadcasted_iota(jnp.int32, (_OUTC, _C), 0)
    c = jax.lax.broadcasted_iota(jnp.int32, (_OUTC, _C), 1)
    return (r == c + 2).astype(jnp.float32)


def _small_placement_t():
    r = jax.lax.broadcasted_iota(jnp.int32, (_OUTC, 8), 0)
    c = jax.lax.broadcasted_iota(jnp.int32, (_OUTC, 8), 1)
    e = (r == c + 82) & (c < 5)
    e = e | ((c == 5) & (r == 0)) | ((c == 6) & (r == 1))
    return e.astype(jnp.float32)


def _body(c0, c1, c2, c3, c4, b0, b1, b2, b3, b4, t0, t1, t2, t3, t4,
          out_ref, small_ref):
    cls_refs = [c0, c1, c2, c3, c4]
    bbox_refs = [b0, b1, b2, b3, b4]
    ctr_refs = [t0, t1, t2, t3, t4]
    dt = (((1,), (1,)), ((), ()))
    ds = (((1,), (0,)), ((), ()))
    small_ref[7:8, :] = jnp.zeros((1, _NLOC[0]), jnp.float32)
    for i in range(_IPB):
      for l in range(5):
        m = _NLOC[l]
        hw = _HWS[l]
        v = cls_refs[l][i]
        ind = (v > _THRESHOLD).astype(jnp.float32)
        srow = jax.lax.dot_general(jnp.ones((1, _C), jnp.float32), ind, dt,
                                   preferred_element_type=jnp.float32)
        mask = (srow > 0.0).astype(jnp.float32)
        big = jax.lax.dot_general(_cls_placement(), v, dt,
                                  preferred_element_type=jnp.float32)
        im = jax.lax.broadcasted_iota(jnp.int32, (1, m), 1)
        small_ref[0:4, 0:m] = bbox_refs[l][i]
        small_ref[4:5, 0:m] = ctr_refs[l][i]
        small_ref[5:6, 0:m] = ((im % hw) * _STRIDES[l]).astype(jnp.float32)
        small_ref[6:7, 0:m] = ((im // hw) * _STRIDES[l]).astype(jnp.float32)
        t = jax.lax.dot_general(_small_placement_t(), small_ref[:, 0:m], ds,
                                preferred_element_type=jnp.float32)
        out_ref[i, :, pl.ds(_OFFS[l], m)] = (big + t) * mask


def kernel(cls_scores_0, cls_scores_1, cls_scores_2, cls_scores_3, cls_scores_4,
           bbox_preds_0, bbox_preds_1, bbox_preds_2, bbox_preds_3, bbox_preds_4,
           centernesses_0, centernesses_1, centernesses_2, centernesses_3,
           centernesses_4):
    n = cls_scores_0.shape[0]
    cls_l = [cls_scores_0, cls_scores_1, cls_scores_2, cls_scores_3, cls_scores_4]
    bbox_l = [bbox_preds_0, bbox_preds_1, bbox_preds_2, bbox_preds_3, bbox_preds_4]
    ctr_l = [centernesses_0, centernesses_1, centernesses_2, centernesses_3,
             centernesses_4]
    args = []
    specs = []
    for l in range(5):
        m = _NLOC[l]
        args.append(jnp.transpose(cls_l[l], (0, 2, 3, 1)).reshape(n, m, _C))
        specs.append(pl.BlockSpec((_IPB, m, _C), lambda i: (i, 0, 0)))
    for lst, ch in ((bbox_l, 4), (ctr_l, 1)):
        for l in range(5):
            args.append(lst[l].reshape(n, ch, _NLOC[l]))
            specs.append(pl.BlockSpec((_IPB, ch, _NLOC[l]),
                                      lambda i: (i, 0, 0)))
    out = pl.pallas_call(
        _body,
        grid=(n // _IPB,),
        in_specs=specs,
        out_specs=pl.BlockSpec((_IPB, _OUTC, _TOT), lambda i: (i, 0, 0)),
        out_shape=jax.ShapeDtypeStruct((n, _OUTC, _TOT), jnp.float32),
        scratch_shapes=[pltpu.VMEM((8, _NLOC[0]), jnp.float32)],
        compiler_params=pltpu.CompilerParams(
            dimension_semantics=("parallel",)),
    )(*args)
    return jnp.transpose(out, (0, 2, 1))


# speedup vs baseline: 1.0469x; 1.0017x over previous
import jax
import jax.numpy as jnp
from jax.experimental import pallas as pl
from jax.experimental.pallas import tpu as pltpu

_STRIDES = [8, 16, 32, 64, 128]
_THRESHOLD = 0.99
_HWS = [64, 32, 16, 8, 4]
_NLOC = [hw * hw for hw in _HWS]
_OFFS = [0, 4096, 5120, 5376, 5440]
_TOT = 5456
_C = 80
_OUTC = 87
_IPB = 4


def _cls_placement():
    r = jax.lax.broadcasted_iota(jnp.int32, (_OUTC, _C), 0)
    c = jax.lax.broadcasted_iota(jnp.int32, (_OUTC, _C), 1)
    return (r == c + 2).astype(jnp.float32)


def _small_placement_t():
    r = jax.lax.broadcasted_iota(jnp.int32, (_OUTC, 8), 0)
    c = jax.lax.broadcasted_iota(jnp.int32, (_OUTC, 8), 1)
    e = (r == c + 82) & (c < 5)
    e = e | ((c == 5) & (r == 0)) | ((c == 6) & (r == 1))
    return e.astype(jnp.float32)


def _body(c0, c1, c2, c3, c4, b0, b1, b2, b3, b4, t0, t1, t2, t3, t4,
          out_ref, small_ref):
    cls_refs = [c0, c1, c2, c3, c4]
    bbox_refs = [b0, b1, b2, b3, b4]
    ctr_refs = [t0, t1, t2, t3, t4]
    dt = (((1,), (1,)), ((), ()))
    ds = (((1,), (0,)), ((), ()))
    small_ref[7:8, :] = jnp.zeros((1, _NLOC[0]), jnp.float32)
    for i in range(_IPB):
      for l in range(5):
        m = _NLOC[l]
        hw = _HWS[l]
        v = cls_refs[l][i]
        ind = (v > _THRESHOLD).astype(jnp.float32)
        srow = jax.lax.dot_general(jnp.ones((1, _C), jnp.float32), ind, dt,
                                   preferred_element_type=jnp.float32)
        mask = (srow > 0.0).astype(jnp.float32)
        big = jax.lax.dot_general(_cls_placement(), v, dt,
                                  preferred_element_type=jnp.float32)
        im = jax.lax.broadcasted_iota(jnp.int32, (1, m), 1)
        small_ref[0:4, 0:m] = bbox_refs[l][i]
        small_ref[4:5, 0:m] = ctr_refs[l][i]
        small_ref[5:6, 0:m] = ((im % hw) * _STRIDES[l]).astype(jnp.float32)
        small_ref[6:7, 0:m] = ((im // hw) * _STRIDES[l]).astype(jnp.float32)
        t = jax.lax.dot_general(_small_placement_t(), small_ref[:, 0:m], ds,
                                preferred_element_type=jnp.float32)
        out_ref[i, :, pl.ds(_OFFS[l], m)] = (big + t) * mask


def kernel(cls_scores_0, cls_scores_1, cls_scores_2, cls_scores_3, cls_scores_4,
           bbox_preds_0, bbox_preds_1, bbox_preds_2, bbox_preds_3, bbox_preds_4,
           centernesses_0, centernesses_1, centernesses_2, centernesses_3,
           centernesses_4):
    n = cls_scores_0.shape[0]
    cls_l = [cls_scores_0, cls_scores_1, cls_scores_2, cls_scores_3, cls_scores_4]
    bbox_l = [bbox_preds_0, bbox_preds_1, bbox_preds_2, bbox_preds_3, bbox_preds_4]
    ctr_l = [centernesses_0, centernesses_1, centernesses_2, centernesses_3,
             centernesses_4]
    args = []
    specs = []
    for l in range(5):
        m = _NLOC[l]
        args.append(jnp.transpose(cls_l[l], (0, 2, 3, 1)).reshape(n, m, _C))
        specs.append(pl.BlockSpec((_IPB, m, _C), lambda i: (i, 0, 0)))
    for lst, ch in ((bbox_l, 4), (ctr_l, 1)):
        for l in range(5):
            args.append(lst[l].reshape(n, ch, _NLOC[l]))
            specs.append(pl.BlockSpec((_IPB, ch, _NLOC[l]),
                                      lambda i: (i, 0, 0)))
    out = pl.pallas_call(
        _body,
        grid=(n // _IPB,),
        in_specs=specs,
        out_specs=pl.BlockSpec((_IPB, _OUTC, _TOT), lambda i: (i, 0, 0)),
        out_shape=jax.ShapeDtypeStruct((n, _OUTC, _TOT), jnp.float32),
        scratch_shapes=[pltpu.VMEM((8, _NLOC[0]), jnp.float32)],
        compiler_params=pltpu.CompilerParams(
            dimension_semantics=("parallel",)),
    )(*args)
    return jnp.transpose(out, (0, 2, 1))
